# baseline (speedup 1.0000x reference)
import functools

import jax
import jax.numpy as jnp
from jax import lax
from jax.experimental import pallas as pl
from jax.experimental.pallas import tpu as pltpu
from jax.experimental.pallas import tpu_sc as plsc

B = 4096
SEQ = 200
E = 64
C = 100
NC = 2
NS = 16
NW = NC * NS
BPW = B // NW
CHUNK = 100
NCHUNK = BPW * (SEQ // CHUNK)
NBUF = 4
NSUPER = NCHUNK // NBUF
LANES = 16


@functools.partial(
    pl.kernel,
    out_type=jax.ShapeDtypeStruct((B, E), jnp.float32),
    mesh=plsc.VectorSubcoreMesh(core_axis_name="c", subcore_axis_name="s"),
    scratch_types=[
        pltpu.VMEM((NCHUNK, CHUNK), jnp.int32),
        pltpu.VMEM((NBUF, CHUNK, E), jnp.float32),
        pltpu.VMEM((BPW, E), jnp.float32),
        pltpu.SemaphoreType.DMA,
        pltpu.SemaphoreType.DMA,
        pltpu.SemaphoreType.DMA,
        pltpu.SemaphoreType.DMA,
    ],
    compiler_params=pltpu.CompilerParams(use_tc_tiling_on_sc=False),
)
def _sc_pool(ids_hbm, table_hbm, out_hbm, idx_v, bufs, pooled_v,
             sem0, sem1, sem2, sem3):
    sems = (sem0, sem1, sem2, sem3)
    wid = lax.axis_index("s") * NC + lax.axis_index("c")
    pltpu.sync_copy(ids_hbm.at[wid], idx_v)

    def start(c, k):
        pltpu.make_async_copy(
            table_hbm.at[idx_v.at[c]], bufs.at[k], sems[k]).start()

    def wait(c, k):
        pltpu.make_async_copy(
            table_hbm.at[idx_v.at[c]], bufs.at[k], sems[k]).wait()

    def reduce_chunk(k, acc):
        buf = bufs.at[k]

        def body(j, a):
            return (a[0] + buf[j, pl.ds(0, LANES)],
                    a[1] + buf[j, pl.ds(LANES, LANES)],
                    a[2] + buf[j, pl.ds(2 * LANES, LANES)],
                    a[3] + buf[j, pl.ds(3 * LANES, LANES)])

        return lax.fori_loop(0, CHUNK, body, acc, unroll=4)

    for k in range(NBUF):
        start(k, k)

    zero = jnp.zeros((LANES,), jnp.float32)

    def super_body(s, carry):
        c0 = NBUF * s
        for half in range(2):
            acc = (zero, zero, zero, zero)
            for kk in range(2):
                k = 2 * half + kk
                c = c0 + k
                wait(c, k)

                @pl.when(s < NSUPER - 1)
                def _():
                    start(c + NBUF, k)

                acc = reduce_chunk(k, acc)
            r = 2 * s + half
            pooled_v[r, pl.ds(0, LANES)] = acc[0]
            pooled_v[r, pl.ds(LANES, LANES)] = acc[1]
            pooled_v[r, pl.ds(2 * LANES, LANES)] = acc[2]
            pooled_v[r, pl.ds(3 * LANES, LANES)] = acc[3]
        return carry

    lax.fori_loop(0, NSUPER, super_body, 0)
    pltpu.sync_copy(pooled_v, out_hbm.at[pl.ds(wid * BPW, BPW)])


def _tc_head_body(sums_ref, w_ref, b_ref, labels_ref, loss_ref, logits_ref):
    x = sums_ref[...] * (1.0 / SEQ)
    logits = lax.dot_general(
        x, w_ref[...], (((1,), (1,)), ((), ())),
        preferred_element_type=jnp.float32) + b_ref[...]
    logits_ref[...] = logits
    m = jnp.max(logits, axis=1, keepdims=True)
    logz = jnp.log(jnp.sum(jnp.exp(logits - m), axis=1, keepdims=True)) + m
    cols = lax.broadcasted_iota(jnp.int32, (B, C), 1)
    gold = jnp.sum(jnp.where(cols == labels_ref[...], logits, 0.0),
                   axis=1, keepdims=True)
    loss_ref[0, 0] = jnp.mean(logz - gold)


_tc_head = pl.pallas_call(
    _tc_head_body,
    out_shape=(jax.ShapeDtypeStruct((1, 1), jnp.float32),
               jax.ShapeDtypeStruct((B, C), jnp.float32)),
    out_specs=(pl.BlockSpec(memory_space=pltpu.SMEM),
               pl.BlockSpec(memory_space=pltpu.VMEM)),
)


def kernel(emb_table, W, b, input_ids, labels):
    ids = input_ids.astype(jnp.int32).reshape(NW, NCHUNK, CHUNK)
    sums = _sc_pool(ids, emb_table)
    loss2, logits = _tc_head(sums, W, b.reshape(1, C),
                             labels.astype(jnp.int32).reshape(B, 1))
    return (loss2[0, 0], logits)
